# Initial kernel scaffold; baseline (speedup 1.0000x reference)
import jax
import jax.numpy as jnp
from jax.experimental import pallas as pl


def kernel(input, filter, weight):
    raise NotImplementedError("write your pallas kernel here")


# speedup vs baseline: 1.8791x; 1.8791x over previous
import functools

import jax
import jax.numpy as jnp
import numpy as np
from jax import lax
from jax.experimental import pallas as pl
from jax.experimental.pallas import tpu as pltpu
from jax.experimental.pallas import tpu_sc as plsc

_NUM_ORTHOS = 100
_DIMS = 128
_NC = 2
_NS = 16
_NW = _NC * _NS
_LANES = 16
_GROUP = 64
_TILE = _GROUP * _DIMS

def _sc_body(n_per_w, x_hbm, c_hbm, out_hbm, x_v, c_v, ob0, ob1, sem0, sem1):
    wid = lax.axis_index("s") * _NC + lax.axis_index("c")
    base = wid * n_per_w
    pltpu.sync_copy(x_hbm.at[pl.ds(base, n_per_w)], x_v)
    pltpu.sync_copy(c_hbm, c_v)

    lanes = lax.iota(jnp.int32, _LANES)
    n_groups = n_per_w // _GROUP

    @pl.loop(0, n_groups, step=2)
    def _outer(gp):
        for b in range(2):
            ob, sem = (ob0, sem0) if b == 0 else (ob1, sem1)
            g = gp + b

            @pl.when(g >= 2)
            def _():
                pltpu.make_async_copy(ob, out_hbm.at[pl.ds(0, _TILE)], sem).wait()

            for j in range(_GROUP // _LANES):
                x = x_v[pl.ds(g * _GROUP + j * _LANES, _LANES)]
                v = jnp.clip(x * 99.0, 0.0, 99.0)
                s = jnp.minimum(v.astype(jnp.int32), 98)
                frac = v - s.astype(jnp.float32)
                row = s * _DIMS
                obase = lanes * _DIMS + j * (_LANES * _DIMS)

                @pl.loop(0, _DIMS, unroll=8)
                def _inner(d):
                    c = plsc.load_gather(c_v, [row + d])
                    out = jnp.where((frac <= jnp.abs(c)) != (c < 0.0),
                                    1.0, -1.0).astype(jnp.float32)
                    plsc.store_scatter(ob, [obase + d], out)

            pltpu.async_copy(
                ob, out_hbm.at[pl.ds((base + g * _GROUP) * _DIMS, _TILE)], sem)

    for ob, sem in ((ob0, sem0), (ob1, sem1)):
        pltpu.make_async_copy(ob, out_hbm.at[pl.ds(0, _TILE)], sem).wait()


@jax.jit
def kernel(input, filter, weight):
    shape = input.shape
    n = input.size
    dims = weight.shape[1]
    n_per_w = n // _NW

    ws, we = weight[:-1], weight[1:]
    t = jnp.where(ws == we, jnp.float32(2.0),
                  jnp.maximum(filter, jnp.float32(1e-38)))
    c = (t * ws).reshape(-1)

    x = input.reshape(-1)

    mesh = plsc.VectorSubcoreMesh(
        core_axis_name="c", subcore_axis_name="s",
        num_cores=_NC, num_subcores=_NS)
    fn = pl.kernel(
        functools.partial(_sc_body, n_per_w),
        out_type=jax.ShapeDtypeStruct((n * dims,), jnp.float32),
        mesh=mesh,
        compiler_params=pltpu.CompilerParams(needs_layout_passes=False),
        scratch_types=[
            pltpu.VMEM((n_per_w,), jnp.float32),
            pltpu.VMEM(((_NUM_ORTHOS - 1) * dims,), jnp.float32),
            pltpu.VMEM((_TILE,), jnp.float32),
            pltpu.VMEM((_TILE,), jnp.float32),
            pltpu.SemaphoreType.DMA,
            pltpu.SemaphoreType.DMA,
        ],
    )
    out = fn(x, c)
    return out.reshape(shape + (dims,))


# speedup vs baseline: 4.1939x; 2.2319x over previous
import functools

import jax
import jax.numpy as jnp
import numpy as np
from jax import lax
from jax.experimental import pallas as pl
from jax.experimental.pallas import tpu as pltpu
from jax.experimental.pallas import tpu_sc as plsc

_NUM_ORTHOS = 100
_DIMS = 128
_NC = 2
_NS = 16
_NW = _NC * _NS
_LANES = 16
_GROUP = 128
_TILE = _GROUP * _DIMS
_FSTRIDE = 17
_CPAD = _DIMS + 8
_NBUF = 2


def _sc_body(n_per_w, x_hbm, c_hbm, out_hbm, x_v, c_v, *rest):
    rss = rest[:_NBUF]
    fss = rest[_NBUF:2 * _NBUF]
    obs = rest[2 * _NBUF:3 * _NBUF]
    osems = rest[3 * _NBUF:4 * _NBUF]

    wid = lax.axis_index("s") * _NC + lax.axis_index("c")
    base = wid * n_per_w
    pltpu.sync_copy(x_hbm.at[pl.ds(base, n_per_w)], x_v)
    pltpu.sync_copy(c_hbm, c_v)

    lanes = lax.iota(jnp.int32, _LANES)
    n_groups = n_per_w // _GROUP

    @pl.loop(0, n_groups, step=_NBUF)
    def _outer(gp):
        for b in range(_NBUF):
            rs_b, fs_b, ob_b, osem_b = rss[b], fss[b], obs[b], osems[b]
            g = gp + b

            @pl.when(g >= _NBUF)
            def _():
                pltpu.make_async_copy(
                    ob_b, out_hbm.at[pl.ds(0, _TILE)], osem_b).wait()

            for j in range(_GROUP // _LANES):
                x = x_v[pl.ds(g * _GROUP + j * _LANES, _LANES)]
                v = jnp.clip(x * 99.0, 0.0, 99.0)
                s = jnp.minimum(v.astype(jnp.int32), 98)
                frac = v - s.astype(jnp.float32)
                row = s * _CPAD
                scbase = (lanes + j * _LANES) * _FSTRIDE
                for i in range(_LANES):
                    plsc.store_scatter(rs_b, [scbase + i], row)
                    plsc.store_scatter(fs_b, [scbase + i], frac)

            @pl.loop(0, _GROUP, unroll=2)
            def _elems(e):
                rowv = rs_b[pl.ds(e * _FSTRIDE, _LANES)]
                frv = fs_b[pl.ds(e * _FSTRIDE, _LANES)]
                for k in range(_DIMS // _LANES):
                    c = plsc.load_gather(c_v, [rowv + (k * _LANES + lanes)])
                    out = jnp.where((frv <= jnp.abs(c)) != (c < 0.0),
                                    1.0, -1.0).astype(jnp.float32)
                    ob_b[pl.ds(e * _DIMS + k * _LANES, _LANES)] = out

            pltpu.async_copy(
                ob_b,
                out_hbm.at[pl.ds((base + g * _GROUP) * _DIMS, _TILE)], osem_b)

    for b in range(_NBUF):
        pltpu.make_async_copy(
            obs[b], out_hbm.at[pl.ds(0, _TILE)], osems[b]).wait()


@jax.jit
def kernel(input, filter, weight):
    shape = input.shape
    n = input.size
    dims = weight.shape[1]
    n_per_w = n // _NW

    ws, we = weight[:-1], weight[1:]
    t = jnp.where(ws == we, jnp.float32(2.0),
                  jnp.maximum(filter, jnp.float32(1e-38)))
    c = jnp.pad(t * ws, ((0, 0), (0, _CPAD - dims))).reshape(-1)

    x = input.reshape(-1)

    mesh = plsc.VectorSubcoreMesh(
        core_axis_name="c", subcore_axis_name="s",
        num_cores=_NC, num_subcores=_NS)
    fn = pl.kernel(
        functools.partial(_sc_body, n_per_w),
        out_type=jax.ShapeDtypeStruct((n * dims,), jnp.float32),
        mesh=mesh,
        compiler_params=pltpu.CompilerParams(needs_layout_passes=False),
        scratch_types=(
            [pltpu.VMEM((n_per_w,), jnp.float32),
             pltpu.VMEM(((_NUM_ORTHOS - 1) * _CPAD,), jnp.float32)]
            + [pltpu.VMEM((_GROUP * _FSTRIDE,), jnp.int32)
               for _ in range(_NBUF)]
            + [pltpu.VMEM((_GROUP * _FSTRIDE,), jnp.float32)
               for _ in range(_NBUF)]
            + [pltpu.VMEM((_TILE,), jnp.float32) for _ in range(_NBUF)]
            + [pltpu.SemaphoreType.DMA for _ in range(_NBUF)]
        ),
    )
    out = fn(x, c)
    return out.reshape(shape + (dims,))


# speedup vs baseline: 6.6014x; 1.5740x over previous
import functools

import jax
import jax.numpy as jnp
import numpy as np
from jax import lax
from jax.experimental import pallas as pl
from jax.experimental.pallas import tpu as pltpu
from jax.experimental.pallas import tpu_sc as plsc

_NUM_ORTHOS = 100
_DIMS = 128
_NC = 2
_NS = 16
_NW = _NC * _NS
_LANES = 16
_GROUP = 128
_TILE = _GROUP * _DIMS
_FSTRIDE = 17
_CPAD = _DIMS + 8
_NBUF = 2


def _sc_body(n_per_w, x_hbm, c_hbm, out_hbm, x_v, c_v, *rest):
    rss = rest[:_NBUF]
    fss = rest[_NBUF:2 * _NBUF]
    obs = rest[2 * _NBUF:3 * _NBUF]
    osems = rest[3 * _NBUF:4 * _NBUF]

    wid = lax.axis_index("s") * _NC + lax.axis_index("c")
    base = wid * n_per_w
    pltpu.sync_copy(x_hbm.at[pl.ds(base, n_per_w)], x_v)
    pltpu.sync_copy(c_hbm, c_v)

    lanes = lax.iota(jnp.int32, _LANES)
    offs = [k * _LANES + lanes for k in range(_DIMS // _LANES)]
    n_groups = n_per_w // _GROUP

    @pl.loop(0, n_groups, step=_NBUF)
    def _outer(gp):
        for b in range(_NBUF):
            rs_b, fs_b, ob_b, osem_b = rss[b], fss[b], obs[b], osems[b]
            g = gp + b

            @pl.when(g >= _NBUF)
            def _():
                pltpu.make_async_copy(
                    ob_b, out_hbm.at[pl.ds(0, _TILE)], osem_b).wait()

            for j in range(_GROUP // _LANES):
                x = x_v[pl.ds(g * _GROUP + j * _LANES, _LANES)]
                v = jnp.clip(x * 99.0, 0.0, 99.0)
                s = jnp.minimum(v.astype(jnp.int32), 98)
                frac = v - s.astype(jnp.float32)
                row = s * _CPAD
                scbase = (lanes + j * _LANES) * _FSTRIDE
                for i in range(_LANES):
                    plsc.store_scatter(rs_b, [scbase + i], row)
                    plsc.store_scatter(fs_b, [scbase + i], frac)

            @pl.loop(0, _GROUP, unroll=2)
            def _elems(e):
                rowv = rs_b[pl.ds(e * _FSTRIDE, _LANES)]
                frv = fs_b[pl.ds(e * _FSTRIDE, _LANES)]
                cs = [plsc.load_gather(c_v, [rowv + offs[k]])
                      for k in range(_DIMS // _LANES)]
                for k in range(_DIMS // _LANES):
                    c = cs[k]
                    out = jnp.where((frv <= jnp.abs(c)) != (c < 0.0),
                                    1.0, -1.0).astype(jnp.float32)
                    ob_b[pl.ds(e * _DIMS + k * _LANES, _LANES)] = out

            pltpu.async_copy(
                ob_b,
                out_hbm.at[pl.ds((base + g * _GROUP) * _DIMS, _TILE)], osem_b)

    for b in range(_NBUF):
        pltpu.make_async_copy(
            obs[b], out_hbm.at[pl.ds(0, _TILE)], osems[b]).wait()


@jax.jit
def kernel(input, filter, weight):
    shape = input.shape
    n = input.size
    dims = weight.shape[1]
    n_per_w = n // _NW

    ws, we = weight[:-1], weight[1:]
    t = jnp.where(ws == we, jnp.float32(2.0),
                  jnp.maximum(filter, jnp.float32(1e-38)))
    c = jnp.pad(t * ws, ((0, 0), (0, _CPAD - dims))).reshape(-1)

    x = input.reshape(-1)

    mesh = plsc.VectorSubcoreMesh(
        core_axis_name="c", subcore_axis_name="s",
        num_cores=_NC, num_subcores=_NS)
    fn = pl.kernel(
        functools.partial(_sc_body, n_per_w),
        out_type=jax.ShapeDtypeStruct((n * dims,), jnp.float32),
        mesh=mesh,
        compiler_params=pltpu.CompilerParams(needs_layout_passes=False),
        scratch_types=(
            [pltpu.VMEM((n_per_w,), jnp.float32),
             pltpu.VMEM(((_NUM_ORTHOS - 1) * _CPAD,), jnp.float32)]
            + [pltpu.VMEM((_GROUP * _FSTRIDE,), jnp.int32)
               for _ in range(_NBUF)]
            + [pltpu.VMEM((_GROUP * _FSTRIDE,), jnp.float32)
               for _ in range(_NBUF)]
            + [pltpu.VMEM((_TILE,), jnp.float32) for _ in range(_NBUF)]
            + [pltpu.SemaphoreType.DMA for _ in range(_NBUF)]
        ),
    )
    out = fn(x, c)
    return out.reshape(shape + (dims,))


# speedup vs baseline: 6.6198x; 1.0028x over previous
import functools

import jax
import jax.numpy as jnp
import numpy as np
from jax import lax
from jax.experimental import pallas as pl
from jax.experimental.pallas import tpu as pltpu
from jax.experimental.pallas import tpu_sc as plsc

_NUM_ORTHOS = 100
_DIMS = 128
_NC = 2
_NS = 16
_NW = _NC * _NS
_LANES = 16
_GROUP = 128
_TILE = _GROUP * _DIMS
_FSTRIDE = 17
_CPAD = _DIMS + 8
_NBUF = 4


def _sc_body(n_per_w, x_hbm, c_hbm, out_hbm, x_v, c_v, *rest):
    rss = rest[:_NBUF]
    fss = rest[_NBUF:2 * _NBUF]
    obs = rest[2 * _NBUF:3 * _NBUF]
    osems = rest[3 * _NBUF:4 * _NBUF]

    wid = lax.axis_index("s") * _NC + lax.axis_index("c")
    base = wid * n_per_w
    pltpu.sync_copy(x_hbm.at[pl.ds(base, n_per_w)], x_v)
    pltpu.sync_copy(c_hbm, c_v)

    lanes = lax.iota(jnp.int32, _LANES)
    offs = [k * _LANES + lanes for k in range(_DIMS // _LANES)]
    n_groups = n_per_w // _GROUP

    @pl.loop(0, n_groups, step=_NBUF)
    def _outer(gp):
        for b in range(_NBUF):
            rs_b, fs_b, ob_b, osem_b = rss[b], fss[b], obs[b], osems[b]
            g = gp + b

            @pl.when(g >= _NBUF)
            def _():
                pltpu.make_async_copy(
                    ob_b, out_hbm.at[pl.ds(0, _TILE)], osem_b).wait()

            for j in range(_GROUP // _LANES):
                x = x_v[pl.ds(g * _GROUP + j * _LANES, _LANES)]
                v = jnp.clip(x * 99.0, 0.0, 99.0)
                s = jnp.minimum(v.astype(jnp.int32), 98)
                frac = v - s.astype(jnp.float32)
                row = s * _CPAD
                scbase = (lanes + j * _LANES) * _FSTRIDE
                for i in range(_LANES):
                    plsc.store_scatter(rs_b, [scbase + i], row)
                    plsc.store_scatter(fs_b, [scbase + i], frac)

            @pl.loop(0, _GROUP, unroll=4)
            def _elems(e):
                rowv = rs_b[pl.ds(e * _FSTRIDE, _LANES)]
                frv = fs_b[pl.ds(e * _FSTRIDE, _LANES)]
                cs = [plsc.load_gather(c_v, [rowv + offs[k]])
                      for k in range(_DIMS // _LANES)]
                for k in range(_DIMS // _LANES):
                    c = cs[k]
                    out = jnp.where((frv <= jnp.abs(c)) != (c < 0.0),
                                    1.0, -1.0).astype(jnp.float32)
                    ob_b[pl.ds(e * _DIMS + k * _LANES, _LANES)] = out

            pltpu.async_copy(
                ob_b,
                out_hbm.at[pl.ds((base + g * _GROUP) * _DIMS, _TILE)], osem_b)

    for b in range(_NBUF):
        pltpu.make_async_copy(
            obs[b], out_hbm.at[pl.ds(0, _TILE)], osems[b]).wait()


@jax.jit
def kernel(input, filter, weight):
    shape = input.shape
    n = input.size
    dims = weight.shape[1]
    n_per_w = n // _NW

    ws, we = weight[:-1], weight[1:]
    t = jnp.where(ws == we, jnp.float32(2.0),
                  jnp.maximum(filter, jnp.float32(1e-38)))
    c = jnp.pad(t * ws, ((0, 0), (0, _CPAD - dims))).reshape(-1)

    x = input.reshape(-1)

    mesh = plsc.VectorSubcoreMesh(
        core_axis_name="c", subcore_axis_name="s",
        num_cores=_NC, num_subcores=_NS)
    fn = pl.kernel(
        functools.partial(_sc_body, n_per_w),
        out_type=jax.ShapeDtypeStruct((n * dims,), jnp.float32),
        mesh=mesh,
        compiler_params=pltpu.CompilerParams(needs_layout_passes=False),
        scratch_types=(
            [pltpu.VMEM((n_per_w,), jnp.float32),
             pltpu.VMEM(((_NUM_ORTHOS - 1) * _CPAD,), jnp.float32)]
            + [pltpu.VMEM((_GROUP * _FSTRIDE,), jnp.int32)
               for _ in range(_NBUF)]
            + [pltpu.VMEM((_GROUP * _FSTRIDE,), jnp.float32)
               for _ in range(_NBUF)]
            + [pltpu.VMEM((_TILE,), jnp.float32) for _ in range(_NBUF)]
            + [pltpu.SemaphoreType.DMA for _ in range(_NBUF)]
        ),
    )
    out = fn(x, c)
    return out.reshape(shape + (dims,))
